# baseline (speedup 1.0000x reference)
import functools

import jax
import jax.numpy as jnp
from jax import lax
from jax.experimental import pallas as pl
from jax.experimental.pallas import tpu as pltpu
from jax.experimental.pallas import tpu_sc as plsc

B, C, V, D, H, O = 1024, 50, 100000, 128, 4096, 4096
N_IDX = B * C

SC_CORES = 2
SC_SUBCORES = 16
NW = SC_CORES * SC_SUBCORES
B_PER_W = N_IDX // NW
GATHER_CHUNK = 400


def _sc_gather(emb_table, flat_idx):
    @functools.partial(
        pl.kernel,
        out_type=jax.ShapeDtypeStruct((N_IDX, D), jnp.float32),
        mesh=plsc.VectorSubcoreMesh(core_axis_name="c", subcore_axis_name="s"),
        scratch_types=[
            pltpu.VMEM((B_PER_W,), jnp.int32),
            pltpu.VMEM((GATHER_CHUNK, D), jnp.float32),
            pltpu.SemaphoreType.DMA,
        ],
    )
    def gather_kernel(table_hbm, idx_hbm, out_hbm, idx_v, rows_v, sem):
        wid = lax.axis_index("s") * SC_CORES + lax.axis_index("c")
        base = wid * B_PER_W
        pltpu.sync_copy(idx_hbm.at[pl.ds(base, B_PER_W)], idx_v)

        @pl.loop(0, B_PER_W, step=GATHER_CHUNK)
        def _(off):
            pltpu.async_copy(
                table_hbm.at[idx_v.at[pl.ds(off, GATHER_CHUNK)]], rows_v, sem
            ).wait()
            pltpu.sync_copy(rows_v, out_hbm.at[pl.ds(base + off, GATHER_CHUNK)])

    return gather_kernel(emb_table, flat_idx)


def _fc1_body(x_ref, w_ref, b_ref, o_ref, xb_ref):
    i, c = pl.program_id(0), pl.program_id(1)

    @pl.when(i == 0)
    def _():
        xb_ref[:, pl.ds(c * D, D)] = x_ref[...].astype(jnp.bfloat16)

    @pl.when(c == C - 1)
    def _():
        r = lax.dot_general(
            xb_ref[...],
            w_ref[...],
            (((1,), (0,)), ((), ())),
            preferred_element_type=jnp.float32,
        )
        o_ref[...] = jnp.maximum(r + b_ref[...], 0.0).astype(jnp.bfloat16)


def _fc1(x, w, b, *, bn):
    n = w.shape[1]
    return pl.pallas_call(
        _fc1_body,
        grid=(n // bn, C),
        in_specs=[
            pl.BlockSpec((B, D), lambda i, c: (jnp.where(i == 0, c, C - 1), 0)),
            pl.BlockSpec((C * D, bn), lambda i, c: (0, i)),
            pl.BlockSpec((1, bn), lambda i, c: (0, i)),
        ],
        out_specs=pl.BlockSpec((B, bn), lambda i, c: (0, i)),
        out_shape=jax.ShapeDtypeStruct((B, n), jnp.bfloat16),
        scratch_shapes=[pltpu.VMEM((B, C * D), jnp.bfloat16)],
        compiler_params=pltpu.CompilerParams(
            dimension_semantics=("arbitrary", "arbitrary"),
        ),
    )(x, w, b.reshape(1, n))


def _fc2_body(x_ref, w_ref, b_ref, o_ref):
    r = lax.dot_general(
        x_ref[...],
        w_ref[...],
        (((1,), (0,)), ((), ())),
        preferred_element_type=jnp.float32,
    )
    o_ref[...] = r + b_ref[...]


def _fc2(x, w, b, *, bn):
    m, kdim = x.shape
    n = w.shape[1]
    return pl.pallas_call(
        _fc2_body,
        grid=(n // bn,),
        in_specs=[
            pl.BlockSpec((m, kdim), lambda i: (0, 0)),
            pl.BlockSpec((kdim, bn), lambda i: (0, i)),
            pl.BlockSpec((1, bn), lambda i: (0, i)),
        ],
        out_specs=pl.BlockSpec((m, bn), lambda i: (0, i)),
        out_shape=jax.ShapeDtypeStruct((m, n), jnp.float32),
        compiler_params=pltpu.CompilerParams(
            dimension_semantics=("parallel",),
        ),
    )(x, w, b.reshape(1, n))


def kernel(context_words, emb_table, W1, b1, W2, b2):
    flat_idx = context_words.T.reshape(-1).astype(jnp.int32)
    x = _sc_gather(emb_table, flat_idx)
    h = _fc1(x, W1, b1, bn=512)
    out = _fc2(h, W2, b2, bn=512)
    return out


# speedup vs baseline: 1.2432x; 1.2432x over previous
import functools

import jax
import jax.numpy as jnp
from jax import lax
from jax.experimental import pallas as pl
from jax.experimental.pallas import tpu as pltpu
from jax.experimental.pallas import tpu_sc as plsc

B, C, V, D, H, O = 1024, 50, 100000, 128, 4096, 4096
N_IDX = B * C

SC_CORES = 2
SC_SUBCORES = 16
NW = SC_CORES * SC_SUBCORES
B_PER_W = N_IDX // NW
GATHER_CHUNK = 400


def _sc_gather(emb_table, flat_idx):
    @functools.partial(
        pl.kernel,
        out_type=jax.ShapeDtypeStruct((N_IDX, D), jnp.float32),
        mesh=plsc.VectorSubcoreMesh(core_axis_name="c", subcore_axis_name="s"),
        scratch_types=[
            pltpu.VMEM((B_PER_W,), jnp.int32),
            pltpu.VMEM((GATHER_CHUNK, D), jnp.float32),
            pltpu.SemaphoreType.DMA,
        ],
    )
    def gather_kernel(table_hbm, idx_hbm, out_hbm, idx_v, rows_v, sem):
        wid = lax.axis_index("s") * SC_CORES + lax.axis_index("c")
        base = wid * B_PER_W
        pltpu.sync_copy(idx_hbm.at[pl.ds(base, B_PER_W)], idx_v)

        @pl.loop(0, B_PER_W, step=GATHER_CHUNK)
        def _(off):
            pltpu.async_copy(
                table_hbm.at[idx_v.at[pl.ds(off, GATHER_CHUNK)]], rows_v, sem
            ).wait()
            pltpu.sync_copy(rows_v, out_hbm.at[pl.ds(base + off, GATHER_CHUNK)])

    return gather_kernel(emb_table, flat_idx)


def _fc1_body(x_hbm, w_ref, b_ref, o_ref, xb_ref, sem):
    i = pl.program_id(0)

    @pl.when(i == 0)
    def _():
        copies = [
            pltpu.make_async_copy(
                x_hbm.at[pl.ds(c * B, B), :],
                xb_ref.at[:, pl.ds(c * D, D)],
                sem,
            )
            for c in range(C)
        ]
        for cp in copies:
            cp.start()
        for cp in copies:
            cp.wait()

    r = lax.dot_general(
        xb_ref[...],
        w_ref[...],
        (((1,), (0,)), ((), ())),
        preferred_element_type=jnp.float32,
    )
    o_ref[...] = jnp.maximum(r + b_ref[...], 0.0).astype(jnp.bfloat16)


def _fc1(x, w, b, *, bn):
    n = w.shape[1]
    return pl.pallas_call(
        _fc1_body,
        grid=(n // bn,),
        in_specs=[
            pl.BlockSpec(memory_space=pl.ANY),
            pl.BlockSpec((C * D, bn), lambda i: (0, i)),
            pl.BlockSpec((1, bn), lambda i: (0, i)),
        ],
        out_specs=pl.BlockSpec((B, bn), lambda i: (0, i)),
        out_shape=jax.ShapeDtypeStruct((B, n), jnp.bfloat16),
        scratch_shapes=[
            pltpu.VMEM((B, C * D), jnp.float32),
            pltpu.SemaphoreType.DMA,
        ],
        compiler_params=pltpu.CompilerParams(
            dimension_semantics=("arbitrary",),
        ),
    )(x, w, b.reshape(1, n))


def _fc2_body(x_ref, w_ref, b_ref, o_ref):
    r = lax.dot_general(
        x_ref[...],
        w_ref[...],
        (((1,), (0,)), ((), ())),
        preferred_element_type=jnp.float32,
    )
    o_ref[...] = r + b_ref[...]


def _fc2(x, w, b, *, bn):
    m, kdim = x.shape
    n = w.shape[1]
    return pl.pallas_call(
        _fc2_body,
        grid=(n // bn,),
        in_specs=[
            pl.BlockSpec((m, kdim), lambda i: (0, 0)),
            pl.BlockSpec((kdim, bn), lambda i: (0, i)),
            pl.BlockSpec((1, bn), lambda i: (0, i)),
        ],
        out_specs=pl.BlockSpec((m, bn), lambda i: (0, i)),
        out_shape=jax.ShapeDtypeStruct((m, n), jnp.float32),
        compiler_params=pltpu.CompilerParams(
            dimension_semantics=("parallel",),
        ),
    )(x, w, b.reshape(1, n))


def kernel(context_words, emb_table, W1, b1, W2, b2):
    flat_idx = context_words.T.reshape(-1).astype(jnp.int32)
    x = _sc_gather(emb_table, flat_idx)
    h = _fc1(x, W1, b1, bn=512)
    out = _fc2(h, W2, b2, bn=512)
    return out


# speedup vs baseline: 1.2538x; 1.0085x over previous
import functools

import jax
import jax.numpy as jnp
from jax import lax
from jax.experimental import pallas as pl
from jax.experimental.pallas import tpu as pltpu
from jax.experimental.pallas import tpu_sc as plsc

B, C, V, D, H, O = 1024, 50, 100000, 128, 4096, 4096
N_IDX = B * C

SC_CORES = 2
SC_SUBCORES = 16
NW = SC_CORES * SC_SUBCORES
B_PER_W = N_IDX // NW
GATHER_CHUNK = 400


def _sc_gather(emb_table, flat_idx):
    n_chunks = B_PER_W // GATHER_CHUNK

    @functools.partial(
        pl.kernel,
        out_type=jax.ShapeDtypeStruct((N_IDX, D), jnp.float32),
        mesh=plsc.VectorSubcoreMesh(core_axis_name="c", subcore_axis_name="s"),
        scratch_types=[
            pltpu.VMEM((B_PER_W,), jnp.int32),
            pltpu.VMEM((GATHER_CHUNK, D), jnp.float32),
            pltpu.VMEM((GATHER_CHUNK, D), jnp.float32),
            pltpu.SemaphoreType.DMA,
            pltpu.SemaphoreType.DMA,
            pltpu.SemaphoreType.DMA,
            pltpu.SemaphoreType.DMA,
        ],
    )
    def gather_kernel(table_hbm, idx_hbm, out_hbm, idx_v, r0, r1, sg0, sg1, sw0, sw1):
        wid = lax.axis_index("s") * SC_CORES + lax.axis_index("c")
        base = wid * B_PER_W
        pltpu.sync_copy(idx_hbm.at[pl.ds(base, B_PER_W)], idx_v)

        bufs = [(r0, sg0, sw0), (r1, sg1, sw1)]
        gathers, writes = [], []
        for i in range(n_chunks):
            r, sg, sw = bufs[i % 2]
            gathers.append(pltpu.make_async_copy(
                table_hbm.at[idx_v.at[pl.ds(i * GATHER_CHUNK, GATHER_CHUNK)]],
                r, sg))
            writes.append(pltpu.make_async_copy(
                r, out_hbm.at[pl.ds(base + i * GATHER_CHUNK, GATHER_CHUNK)], sw))

        gathers[0].start()
        for i in range(n_chunks):
            if i + 1 < n_chunks:
                if i >= 1:
                    writes[i - 1].wait()
                gathers[i + 1].start()
            gathers[i].wait()
            writes[i].start()
        writes[n_chunks - 2].wait()
        writes[n_chunks - 1].wait()

    return gather_kernel(emb_table, flat_idx)


def _fc1_body(x_hbm, w_ref, b_ref, o_ref, xb_ref, sem):
    i = pl.program_id(0)

    @pl.when(i == 0)
    def _():
        copies = [
            pltpu.make_async_copy(
                x_hbm.at[pl.ds(c * B, B), :],
                xb_ref.at[:, pl.ds(c * D, D)],
                sem,
            )
            for c in range(C)
        ]
        for cp in copies:
            cp.start()
        for cp in copies:
            cp.wait()

    r = lax.dot_general(
        xb_ref[...],
        w_ref[...],
        (((1,), (0,)), ((), ())),
        preferred_element_type=jnp.float32,
    )
    o_ref[...] = jnp.maximum(r + b_ref[...], 0.0).astype(jnp.bfloat16)


def _fc1(x, w, b, *, bn):
    n = w.shape[1]
    return pl.pallas_call(
        _fc1_body,
        grid=(n // bn,),
        in_specs=[
            pl.BlockSpec(memory_space=pl.ANY),
            pl.BlockSpec((C * D, bn), lambda i: (0, i)),
            pl.BlockSpec((1, bn), lambda i: (0, i)),
        ],
        out_specs=pl.BlockSpec((B, bn), lambda i: (0, i)),
        out_shape=jax.ShapeDtypeStruct((B, n), jnp.bfloat16),
        scratch_shapes=[
            pltpu.VMEM((B, C * D), jnp.float32),
            pltpu.SemaphoreType.DMA,
        ],
        compiler_params=pltpu.CompilerParams(
            dimension_semantics=("arbitrary",),
        ),
    )(x, w, b.reshape(1, n))


def _fc2_body(x_ref, w_ref, b_ref, o_ref):
    r = lax.dot_general(
        x_ref[...],
        w_ref[...],
        (((1,), (0,)), ((), ())),
        preferred_element_type=jnp.float32,
    )
    o_ref[...] = r + b_ref[...]


def _fc2(x, w, b, *, bn):
    m, kdim = x.shape
    n = w.shape[1]
    return pl.pallas_call(
        _fc2_body,
        grid=(n // bn,),
        in_specs=[
            pl.BlockSpec((m, kdim), lambda i: (0, 0)),
            pl.BlockSpec((kdim, bn), lambda i: (0, i)),
            pl.BlockSpec((1, bn), lambda i: (0, i)),
        ],
        out_specs=pl.BlockSpec((m, bn), lambda i: (0, i)),
        out_shape=jax.ShapeDtypeStruct((m, n), jnp.float32),
        compiler_params=pltpu.CompilerParams(
            dimension_semantics=("parallel",),
        ),
    )(x, w, b.reshape(1, n))


def kernel(context_words, emb_table, W1, b1, W2, b2):
    flat_idx = context_words.T.reshape(-1).astype(jnp.int32)
    x = _sc_gather(emb_table, flat_idx)
    h = _fc1(x, W1, b1, bn=512)
    out = _fc2(h, W2, b2, bn=512)
    return out


# speedup vs baseline: 1.2599x; 1.0048x over previous
import functools

import jax
import jax.numpy as jnp
from jax import lax
from jax.experimental import pallas as pl
from jax.experimental.pallas import tpu as pltpu
from jax.experimental.pallas import tpu_sc as plsc

B, C, V, D, H, O = 1024, 50, 100000, 128, 4096, 4096
N_IDX = B * C

SC_CORES = 2
SC_SUBCORES = 16
NW = SC_CORES * SC_SUBCORES
B_PER_W = N_IDX // NW
GATHER_CHUNK = 400


def _sc_gather(emb_table, flat_idx):
    n_chunks = B_PER_W // GATHER_CHUNK

    @functools.partial(
        pl.kernel,
        out_type=jax.ShapeDtypeStruct((N_IDX, D), jnp.float32),
        mesh=plsc.VectorSubcoreMesh(core_axis_name="c", subcore_axis_name="s"),
        scratch_types=[
            pltpu.VMEM((B_PER_W,), jnp.int32),
            pltpu.VMEM((GATHER_CHUNK, D), jnp.float32),
            pltpu.VMEM((GATHER_CHUNK, D), jnp.float32),
            pltpu.SemaphoreType.DMA,
            pltpu.SemaphoreType.DMA,
            pltpu.SemaphoreType.DMA,
            pltpu.SemaphoreType.DMA,
        ],
    )
    def gather_kernel(table_hbm, idx_hbm, out_hbm, idx_v, r0, r1, sg0, sg1, sw0, sw1):
        wid = lax.axis_index("s") * SC_CORES + lax.axis_index("c")
        base = wid * B_PER_W
        pltpu.sync_copy(idx_hbm.at[pl.ds(base, B_PER_W)], idx_v)

        bufs = [(r0, sg0, sw0), (r1, sg1, sw1)]
        gathers, writes = [], []
        for i in range(n_chunks):
            r, sg, sw = bufs[i % 2]
            gathers.append(pltpu.make_async_copy(
                table_hbm.at[idx_v.at[pl.ds(i * GATHER_CHUNK, GATHER_CHUNK)]],
                r, sg))
            writes.append(pltpu.make_async_copy(
                r, out_hbm.at[pl.ds(base + i * GATHER_CHUNK, GATHER_CHUNK)], sw))

        gathers[0].start()
        for i in range(n_chunks):
            if i + 1 < n_chunks:
                if i >= 1:
                    writes[i - 1].wait()
                gathers[i + 1].start()
            gathers[i].wait()
            writes[i].start()
        writes[n_chunks - 2].wait()
        writes[n_chunks - 1].wait()

    return gather_kernel(emb_table, flat_idx)


def _fc1_body(x_hbm, w_ref, b_ref, o_ref, xb_ref, sem):
    i = pl.program_id(0)

    @pl.when(i == 0)
    def _():
        copies = [
            pltpu.make_async_copy(
                x_hbm.at[pl.ds(c * B, B), :],
                xb_ref.at[:, pl.ds(c * D, D)],
                sem,
            )
            for c in range(C)
        ]
        for cp in copies:
            cp.start()
        for cp in copies:
            cp.wait()

    for h in range(2):
        sl = pl.ds(h * (o_ref.shape[1] // 2), o_ref.shape[1] // 2)
        r = lax.dot_general(
            xb_ref[...],
            w_ref[:, sl],
            (((1,), (0,)), ((), ())),
            preferred_element_type=jnp.float32,
        )
        o_ref[:, sl] = jnp.maximum(r + b_ref[:, sl], 0.0).astype(jnp.bfloat16)


def _fc1(x, w, b, *, bn):
    n = w.shape[1]
    return pl.pallas_call(
        _fc1_body,
        grid=(n // bn,),
        in_specs=[
            pl.BlockSpec(memory_space=pl.ANY),
            pl.BlockSpec((C * D, bn), lambda i: (0, i)),
            pl.BlockSpec((1, bn), lambda i: (0, i)),
        ],
        out_specs=pl.BlockSpec((B, bn), lambda i: (0, i)),
        out_shape=jax.ShapeDtypeStruct((B, n), jnp.bfloat16),
        scratch_shapes=[
            pltpu.VMEM((B, C * D), jnp.float32),
            pltpu.SemaphoreType.DMA,
        ],
        compiler_params=pltpu.CompilerParams(
            dimension_semantics=("arbitrary",),
        ),
    )(x, w, b.reshape(1, n))


def _fc2_body(x_ref, w_ref, b_ref, o_ref):
    for h in range(2):
        sl = pl.ds(h * (o_ref.shape[1] // 2), o_ref.shape[1] // 2)
        r = lax.dot_general(
            x_ref[...],
            w_ref[:, sl],
            (((1,), (0,)), ((), ())),
            preferred_element_type=jnp.float32,
        )
        o_ref[:, sl] = r + b_ref[:, sl]


def _fc2(x, w, b, *, bn):
    m, kdim = x.shape
    n = w.shape[1]
    return pl.pallas_call(
        _fc2_body,
        grid=(n // bn,),
        in_specs=[
            pl.BlockSpec((m, kdim), lambda i: (0, 0)),
            pl.BlockSpec((kdim, bn), lambda i: (0, i)),
            pl.BlockSpec((1, bn), lambda i: (0, i)),
        ],
        out_specs=pl.BlockSpec((m, bn), lambda i: (0, i)),
        out_shape=jax.ShapeDtypeStruct((m, n), jnp.float32),
        compiler_params=pltpu.CompilerParams(
            dimension_semantics=("parallel",),
        ),
    )(x, w, b.reshape(1, n))


def kernel(context_words, emb_table, W1, b1, W2, b2):
    flat_idx = context_words.T.reshape(-1).astype(jnp.int32)
    x = _sc_gather(emb_table, flat_idx)
    h = _fc1(x, W1, b1, bn=512)
    out = _fc2(h, W2, b2, bn=512)
    return out
